# baseline (speedup 1.0000x reference)
import functools
import jax
import jax.numpy as jnp
from jax import lax
from jax.experimental import pallas as pl
from jax.experimental.pallas import tpu as pltpu, tpu_sc as plsc

_NC, _NS = 2, 16
_NW = _NC * _NS


def kernel(x, embed_table):
    B, T, D = x.shape
    ROWS = B * T
    RPW = ROWS // _NW
    CH = 8
    NCH = RPW // CH
    CHW = CH * D

    NBUF = 4
    xf = x.reshape(ROWS * D)
    ef = embed_table.reshape(-1)
    mesh = plsc.VectorSubcoreMesh(core_axis_name="c", subcore_axis_name="s")

    @functools.partial(
        pl.kernel,
        out_type=jax.ShapeDtypeStruct((ROWS * D,), jnp.float32),
        mesh=mesh,
        scratch_types=[
            pltpu.VMEM((NBUF, CHW), jnp.float32),
            pltpu.VMEM((NBUF, CHW), jnp.float32),
            pltpu.SemaphoreType.DMA,
            pltpu.SemaphoreType.DMA,
            pltpu.SemaphoreType.DMA,
            pltpu.SemaphoreType.DMA,
            pltpu.SemaphoreType.DMA,
            pltpu.SemaphoreType.DMA,
            pltpu.SemaphoreType.DMA,
            pltpu.SemaphoreType.DMA,
            pltpu.SemaphoreType.DMA,
            pltpu.SemaphoreType.DMA,
            pltpu.SemaphoreType.DMA,
            pltpu.SemaphoreType.DMA,
        ],
    )
    def k(x_hbm, e_hbm, o_hbm, xb, eb,
          gx0, gx1, gx2, gx3, ge0, ge1, ge2, ge3, so0, so1, so2, so3):
        gx = (gx0, gx1, gx2, gx3)
        ge = (ge0, ge1, ge2, ge3)
        so = (so0, so1, so2, so3)
        w = lax.axis_index("c") * _NS + lax.axis_index("s")
        x0 = w * (RPW * D)
        e0 = lax.rem(w * RPW, T) * D

        gathers = [None] * NBUF
        scatters = [None] * NBUF

        def issue_gathers(c):
            s = c % NBUF
            dx = pltpu.async_copy(
                x_hbm.at[pl.ds(x0 + c * CHW, CHW)], xb.at[s], gx[s]
            )
            de = pltpu.async_copy(
                e_hbm.at[pl.ds(e0 + c * CHW, CHW)], eb.at[s], ge[s]
            )
            gathers[s] = (dx, de)

        for c in range(NBUF - 1):
            issue_gathers(c)
        for c in range(NCH):
            s = c % NBUF
            cn = c + NBUF - 1
            if cn < NCH:
                sn = cn % NBUF
                if scatters[sn] is not None:
                    scatters[sn].wait()
                issue_gathers(cn)
            dx, de = gathers[s]
            dx.wait()
            de.wait()
            xs = xb.at[s]
            es = eb.at[s]

            @plsc.parallel_loop(0, CHW, step=16, unroll=8)
            def body(i):
                xs[pl.ds(i, 16)] = xs[pl.ds(i, 16)] + es[pl.ds(i, 16)]

            scatters[s] = pltpu.async_copy(
                xb.at[s], o_hbm.at[pl.ds(x0 + c * CHW, CHW)], so[s]
            )
        for d in scatters:
            if d is not None:
                d.wait()

    return k(xf, ef).reshape(B, T, D)


# speedup vs baseline: 6.0072x; 6.0072x over previous
import functools

import jax
import jax.numpy as jnp
from jax.experimental import pallas as pl
from jax.experimental.pallas import tpu as pltpu


def _add_kernel(bt, x_ref, emb_ref, o_ref):
    t = pl.program_id(1)
    o_ref[...] = x_ref[...] + emb_ref[pl.ds(t * bt, bt), :]


def kernel(x, embed_table):
    B, T, D = x.shape
    bt = 2048
    grid = (B, T // bt)
    return pl.pallas_call(
        functools.partial(_add_kernel, bt),
        grid=grid,
        in_specs=[
            pl.BlockSpec((1, bt, D), lambda b, t: (b, t, 0)),
            pl.BlockSpec((T, D), lambda b, t: (0, 0)),
        ],
        out_specs=pl.BlockSpec((1, bt, D), lambda b, t: (b, t, 0)),
        out_shape=jax.ShapeDtypeStruct((B, T, D), x.dtype),
        compiler_params=pltpu.CompilerParams(
            dimension_semantics=("parallel", "parallel"),
            vmem_limit_bytes=128 * 1024 * 1024,
        ),
    )(x, embed_table)
